# Initial kernel scaffold; baseline (speedup 1.0000x reference)
import jax
import jax.numpy as jnp
from jax.experimental import pallas as pl


def kernel(x, shape_id, color_id, edge_index, edge_type, batch, shape_emb, color_emb, W_rel1, W_root1, b1, W_rel2, W_root2, b2, lin_W, lin_b):
    raise NotImplementedError("write your pallas kernel here")


# speedup vs baseline: 1.0266x; 1.0266x over previous
import functools

import jax
import jax.numpy as jnp
from jax.experimental import pallas as pl
from jax.experimental.pallas import tpu as pltpu

N_NODES = 50000
N_EDGES = 800000
N_SHAPES = 64
N_COLORS = 32
EMB = 16
HID = 64
N_REL = 3
N_CLS = 4
N_GRAPHS = 512
IN_DIM = 2 * EMB + 1

BN = 5000
NB = N_NODES // BN
F32 = jnp.float32


def _embed_body(sid_ref, cid_ref, x_ref, semb_ref, cemb_ref, out_ref):
    sid = sid_ref[0]
    cid = cid_ref[0]
    iota_s = jax.lax.broadcasted_iota(jnp.int32, (N_SHAPES, BN), 0)
    iota_c = jax.lax.broadcasted_iota(jnp.int32, (N_COLORS, BN), 0)
    oh_s = (iota_s == sid).astype(F32)
    oh_c = (iota_c == cid).astype(F32)
    dn = (((0,), (0,)), ((), ()))
    s = jax.lax.dot_general(oh_s, semb_ref[...], dn, preferred_element_type=F32)
    c = jax.lax.dot_general(oh_c, cemb_ref[...], dn, preferred_element_type=F32)
    pad = jnp.zeros((BN, HID - IN_DIM), dtype=F32)
    out_ref[...] = jnp.concatenate([s, c, x_ref[...], pad], axis=1)


def _layer_body(h_ref, a0_ref, a1_ref, a2_ref, wroot_ref, w0_ref, w1_ref,
                w2_ref, b_ref, out_ref):
    acc = jnp.dot(h_ref[...], wroot_ref[...], preferred_element_type=F32)
    acc += jnp.dot(a0_ref[...], w0_ref[...], preferred_element_type=F32)
    acc += jnp.dot(a1_ref[...], w1_ref[...], preferred_element_type=F32)
    acc += jnp.dot(a2_ref[...], w2_ref[...], preferred_element_type=F32)
    out_ref[...] = jnp.maximum(acc + b_ref[...], 0.0)


def _layer(h, a0, a1, a2, wroot, w0, w1, w2, b2d):
    spec_n = pl.BlockSpec((BN, HID), lambda i: (i, 0))
    spec_w = pl.BlockSpec((HID, HID), lambda i: (0, 0))
    return pl.pallas_call(
        _layer_body,
        grid=(NB,),
        in_specs=[spec_n, spec_n, spec_n, spec_n, spec_w, spec_w, spec_w,
                  spec_w, pl.BlockSpec((1, HID), lambda i: (0, 0))],
        out_specs=spec_n,
        out_shape=jax.ShapeDtypeStruct((N_NODES, HID), F32),
    )(h, a0, a1, a2, wroot, w0, w1, w2, b2d)


def _pool_body(h_ref, batch_ref, linw_ref, linb_ref, out_ref, sums, cnts):
    i = pl.program_id(0)

    @pl.when(i == 0)
    def _():
        sums[...] = jnp.zeros_like(sums)
        cnts[...] = jnp.zeros_like(cnts)

    b = batch_ref[0]
    iota_g = jax.lax.broadcasted_iota(jnp.int32, (N_GRAPHS, BN), 0)
    oh = (iota_g == b).astype(F32)
    dn = (((1,), (0,)), ((), ()))
    sums[...] += jax.lax.dot_general(oh, h_ref[...], dn,
                                     preferred_element_type=F32)
    ones = jnp.ones((BN, 8), dtype=F32)
    cnts[...] += jax.lax.dot_general(oh, ones, dn, preferred_element_type=F32)

    @pl.when(i == NB - 1)
    def _():
        g = sums[...] / jnp.maximum(cnts[:, 0:1], 1.0)
        out_ref[...] = (jnp.dot(g, linw_ref[...], preferred_element_type=F32)
                        + linb_ref[...])


def _pool_final(h, batch, lin_W, lin_b2d):
    batch3 = batch.astype(jnp.int32).reshape(NB, 1, BN)
    return pl.pallas_call(
        _pool_body,
        grid=(NB,),
        in_specs=[
            pl.BlockSpec((BN, HID), lambda i: (i, 0)),
            pl.BlockSpec((1, 1, BN), lambda i: (i, 0, 0)),
            pl.BlockSpec((HID, N_CLS), lambda i: (0, 0)),
            pl.BlockSpec((1, N_CLS), lambda i: (0, 0)),
        ],
        out_specs=pl.BlockSpec((N_GRAPHS, N_CLS), lambda i: (0, 0)),
        out_shape=jax.ShapeDtypeStruct((N_GRAPHS, N_CLS), F32),
        scratch_shapes=[pltpu.VMEM((N_GRAPHS, HID), F32),
                        pltpu.VMEM((N_GRAPHS, 8), F32)],
    )(h, batch3, lin_W, lin_b2d)


def _edge_agg(h, src, dst, edge_type):
    hsrc = jnp.take(h, src, axis=0)
    aggs = []
    for r in range(N_REL):
        m = (edge_type == r).astype(F32)
        cnt = jax.ops.segment_sum(m, dst, num_segments=N_NODES)
        s = jax.ops.segment_sum(hsrc * m[:, None], dst, num_segments=N_NODES)
        aggs.append(s / jnp.maximum(cnt, 1.0)[:, None])
    return aggs


def kernel(x, shape_id, color_id, edge_index, edge_type, batch, shape_emb,
           color_emb, W_rel1, W_root1, b1, W_rel2, W_root2, b2, lin_W, lin_b):
    src = edge_index[0]
    dst = edge_index[1]

    wroot1 = jnp.zeros((HID, HID), F32).at[:IN_DIM].set(W_root1)
    wrel1 = jnp.zeros((N_REL, HID, HID), F32).at[:, :IN_DIM].set(W_rel1)

    sid3 = shape_id.astype(jnp.int32).reshape(NB, 1, BN)
    cid3 = color_id.astype(jnp.int32).reshape(NB, 1, BN)
    h0 = pl.pallas_call(
        _embed_body,
        grid=(NB,),
        in_specs=[
            pl.BlockSpec((1, 1, BN), lambda i: (i, 0, 0)),
            pl.BlockSpec((1, 1, BN), lambda i: (i, 0, 0)),
            pl.BlockSpec((BN, 1), lambda i: (i, 0)),
            pl.BlockSpec((N_SHAPES, EMB), lambda i: (0, 0)),
            pl.BlockSpec((N_COLORS, EMB), lambda i: (0, 0)),
        ],
        out_specs=pl.BlockSpec((BN, HID), lambda i: (i, 0)),
        out_shape=jax.ShapeDtypeStruct((N_NODES, HID), F32),
    )(sid3, cid3, x, shape_emb, color_emb)

    a0, a1, a2 = _edge_agg(h0, src, dst, edge_type)
    h1 = _layer(h0, a0, a1, a2, wroot1, wrel1[0], wrel1[1], wrel1[2],
                b1.reshape(1, HID))

    a0, a1, a2 = _edge_agg(h1, src, dst, edge_type)
    h2 = _layer(h1, a0, a1, a2, W_root2, W_rel2[0], W_rel2[1], W_rel2[2],
                b2.reshape(1, HID))

    return _pool_final(h2, batch, lin_W, lin_b2d=lin_b.reshape(1, N_CLS))


# speedup vs baseline: 5.5719x; 5.4277x over previous
import functools

import jax
import jax.numpy as jnp
from jax import lax
from jax.experimental import pallas as pl
from jax.experimental.pallas import tpu as pltpu
from jax.experimental.pallas import tpu_sc as plsc

N_NODES = 50000
N_EDGES = 800000
N_SHAPES = 64
N_COLORS = 32
EMB = 16
HID = 64
N_REL = 3
N_CLS = 4
N_GRAPHS = 512
IN_DIM = 2 * EMB + 1

BN = 5000
NB = N_NODES // BN
F32 = jnp.float32


def _embed_body(sid_ref, cid_ref, x_ref, semb_ref, cemb_ref, out_ref):
    sid = sid_ref[0]
    cid = cid_ref[0]
    iota_s = jax.lax.broadcasted_iota(jnp.int32, (N_SHAPES, BN), 0)
    iota_c = jax.lax.broadcasted_iota(jnp.int32, (N_COLORS, BN), 0)
    oh_s = (iota_s == sid).astype(F32)
    oh_c = (iota_c == cid).astype(F32)
    dn = (((0,), (0,)), ((), ()))
    s = jax.lax.dot_general(oh_s, semb_ref[...], dn, preferred_element_type=F32)
    c = jax.lax.dot_general(oh_c, cemb_ref[...], dn, preferred_element_type=F32)
    ones = jnp.ones((BN, 1), dtype=F32)
    pad = jnp.zeros((BN, HID - IN_DIM - 1), dtype=F32)
    out_ref[...] = jnp.concatenate([s, c, x_ref[...], ones, pad], axis=1)


def _layer_body(h_ref, a0_ref, a1_ref, a2_ref, inv_ref, wroot_ref, w0_ref,
                w1_ref, w2_ref, b_ref, out_ref):
    acc = jnp.dot(h_ref[...], wroot_ref[...], preferred_element_type=F32)
    inv = inv_ref[...]
    acc += jnp.dot(a0_ref[...] * inv[:, 0:1], w0_ref[...],
                   preferred_element_type=F32)
    acc += jnp.dot(a1_ref[...] * inv[:, 1:2], w1_ref[...],
                   preferred_element_type=F32)
    acc += jnp.dot(a2_ref[...] * inv[:, 2:3], w2_ref[...],
                   preferred_element_type=F32)
    out_ref[...] = jnp.maximum(acc + b_ref[...], 0.0)


def _layer(h, a0, a1, a2, inv, wroot, w0, w1, w2, b2d, da):
    spec_h = pl.BlockSpec((BN, HID), lambda i: (i, 0))
    spec_a = pl.BlockSpec((BN, da), lambda i: (i, 0))
    spec_w = pl.BlockSpec((da, HID), lambda i: (0, 0))
    return pl.pallas_call(
        _layer_body,
        grid=(NB,),
        in_specs=[spec_h, spec_a, spec_a, spec_a,
                  pl.BlockSpec((BN, 8), lambda i: (i, 0)),
                  pl.BlockSpec((HID, HID), lambda i: (0, 0)),
                  spec_w, spec_w, spec_w,
                  pl.BlockSpec((1, HID), lambda i: (0, 0))],
        out_specs=spec_h,
        out_shape=jax.ShapeDtypeStruct((N_NODES, HID), F32),
    )(h, a0, a1, a2, inv, wroot, w0, w1, w2, b2d)


def _pool_body(h_ref, batch_ref, linw_ref, linb_ref, out_ref, sums, cnts):
    i = pl.program_id(0)

    @pl.when(i == 0)
    def _():
        sums[...] = jnp.zeros_like(sums)
        cnts[...] = jnp.zeros_like(cnts)

    b = batch_ref[0]
    iota_g = jax.lax.broadcasted_iota(jnp.int32, (N_GRAPHS, BN), 0)
    oh = (iota_g == b).astype(F32)
    dn = (((1,), (0,)), ((), ()))
    sums[...] += jax.lax.dot_general(oh, h_ref[...], dn,
                                     preferred_element_type=F32)
    ones = jnp.ones((BN, 8), dtype=F32)
    cnts[...] += jax.lax.dot_general(oh, ones, dn, preferred_element_type=F32)

    @pl.when(i == NB - 1)
    def _():
        g = sums[...] / jnp.maximum(cnts[:, 0:1], 1.0)
        out_ref[...] = (jnp.dot(g, linw_ref[...], preferred_element_type=F32)
                        + linb_ref[...])


def _pool_final(h, batch, lin_W, lin_b2d):
    batch3 = batch.astype(jnp.int32).reshape(NB, 1, BN)
    return pl.pallas_call(
        _pool_body,
        grid=(NB,),
        in_specs=[
            pl.BlockSpec((BN, HID), lambda i: (i, 0)),
            pl.BlockSpec((1, 1, BN), lambda i: (i, 0, 0)),
            pl.BlockSpec((HID, N_CLS), lambda i: (0, 0)),
            pl.BlockSpec((1, N_CLS), lambda i: (0, 0)),
        ],
        out_specs=pl.BlockSpec((N_GRAPHS, N_CLS), lambda i: (0, 0)),
        out_shape=jax.ShapeDtypeStruct((N_GRAPHS, N_CLS), F32),
        scratch_shapes=[pltpu.VMEM((N_GRAPHS, HID), F32),
                        pltpu.VMEM((N_GRAPHS, 8), F32)],
    )(h, batch3, lin_W, lin_b2d)


E_PAD = 819200
CH_ROWS = E_PAD // 128
ROWS_PER_TILE = CH_ROWS // 16
SUPERS = ROWS_PER_TILE // 16
ACC_ROWS = N_REL * N_NODES + 16
STRIPE = ACC_ROWS // 16


@functools.lru_cache(maxsize=None)
def _make_agg(sf):
    s2 = sf // 2
    mesh = plsc.VectorSubcoreMesh(core_axis_name="c", subcore_axis_name="s")

    @functools.partial(
        pl.kernel,
        out_type=jax.ShapeDtypeStruct((sf, ACC_ROWS, 8), F32),
        mesh=mesh,
        scratch_types=[
            pltpu.VMEM_SHARED((ACC_ROWS, 8), F32),
            pltpu.VMEM((16, 128), jnp.int32),
            pltpu.VMEM((16, 128), jnp.int32),
            pltpu.VMEM((16, 128), jnp.int32),
            pltpu.VMEM((16, 128, 8), F32),
            pltpu.SemaphoreType.DMA,
            pltpu.SemaphoreType.DMA,
        ],
        compiler_params=pltpu.CompilerParams(use_tc_tiling_on_sc=False),
    )
    def agg_kernel(hview, src2d, slots2d, zeros, out,
                   acc, srcbuf, slotbuf, idxbuf, rows, gsem, ssem):
        c = lax.axis_index("c")
        t = lax.axis_index("s")
        r0 = t * ROWS_PER_TILE
        a0 = t * STRIPE
        for i in range(s2):
            s_id = c * s2 + i
            pltpu.sync_copy(zeros.at[pl.ds(a0, STRIPE)],
                            acc.at[pl.ds(a0, STRIPE)])
            plsc.subcore_barrier()

            def super_body(u, carry):
                row = r0 + u * 16
                pltpu.sync_copy(src2d.at[pl.ds(row, 16)], srcbuf)
                pltpu.sync_copy(slots2d.at[pl.ds(row, 16)], slotbuf)
                for g in range(16):
                    for q in range(8):
                        sl = pl.ds(q * 16, 16)
                        idxbuf[g, sl] = srcbuf[g, sl] * sf + s_id
                for g in range(16):
                    pltpu.async_copy(hview.at[idxbuf.at[g]], rows.at[g], gsem)
                for g in range(16):
                    pltpu.make_async_copy(hview.at[idxbuf.at[g]],
                                          rows.at[g], gsem).wait()
                for g in range(16):
                    pltpu.async_copy(rows.at[g], acc.at[slotbuf.at[g]],
                                     ssem, add=True)
                for g in range(16):
                    pltpu.make_async_copy(rows.at[g],
                                          acc.at[slotbuf.at[g]], ssem).wait()
                return carry

            lax.fori_loop(0, SUPERS, super_body, 0)
            plsc.subcore_barrier()
            pltpu.sync_copy(acc.at[pl.ds(a0, STRIPE)],
                            out.at[s_id].at[pl.ds(a0, STRIPE)])
            plsc.subcore_barrier()

    return agg_kernel


def _edge_aggregate(hview, sf, src2d, slots2d, zeros):
    return _make_agg(sf)(hview, src2d, slots2d, zeros)


def _agg_to_rel_major(agg, sf):
    a = agg[:, :N_REL * N_NODES, :].reshape(sf, N_REL, N_NODES, 8)
    a = a.transpose(1, 2, 0, 3).reshape(N_REL, N_NODES, sf * 8)
    return a[0], a[1], a[2]


def kernel(x, shape_id, color_id, edge_index, edge_type, batch, shape_emb,
           color_emb, W_rel1, W_root1, b1, W_rel2, W_root2, b2, lin_W, lin_b):
    src = edge_index[0].astype(jnp.int32)
    dst = edge_index[1].astype(jnp.int32)
    etype = edge_type.astype(jnp.int32)

    n_pad = E_PAD - N_EDGES
    pad_iota = jnp.arange(n_pad, dtype=jnp.int32)
    src_p = jnp.concatenate([src, pad_iota % N_NODES])
    slots_p = jnp.concatenate([etype * N_NODES + dst,
                               N_REL * N_NODES + (pad_iota % 16)])
    src2d = src_p.reshape(CH_ROWS, 128)
    slots2d = slots_p.reshape(CH_ROWS, 128)
    zeros = jnp.zeros((ACC_ROWS, 8), F32)

    wroot1 = jnp.zeros((HID, HID), F32).at[:IN_DIM].set(W_root1)
    wrel1 = jnp.zeros((N_REL, 48, HID), F32).at[:, :IN_DIM].set(W_rel1)

    sid3 = shape_id.astype(jnp.int32).reshape(NB, 1, BN)
    cid3 = color_id.astype(jnp.int32).reshape(NB, 1, BN)
    h0 = pl.pallas_call(
        _embed_body,
        grid=(NB,),
        in_specs=[
            pl.BlockSpec((1, 1, BN), lambda i: (i, 0, 0)),
            pl.BlockSpec((1, 1, BN), lambda i: (i, 0, 0)),
            pl.BlockSpec((BN, 1), lambda i: (i, 0)),
            pl.BlockSpec((N_SHAPES, EMB), lambda i: (0, 0)),
            pl.BlockSpec((N_COLORS, EMB), lambda i: (0, 0)),
        ],
        out_specs=pl.BlockSpec((BN, HID), lambda i: (i, 0)),
        out_shape=jax.ShapeDtypeStruct((N_NODES, HID), F32),
    )(sid3, cid3, x, shape_emb, color_emb)

    h0v = h0[:, :48].reshape(N_NODES * 6, 8)
    agg1 = _edge_aggregate(h0v, 6, src2d, slots2d, zeros)
    a0, a1, a2 = _agg_to_rel_major(agg1, 6)
    cnt = agg1[4, :N_REL * N_NODES, 1].reshape(N_REL, N_NODES)
    inv = (1.0 / jnp.maximum(cnt, 1.0)).T
    inv8 = jnp.zeros((N_NODES, 8), F32).at[:, :N_REL].set(inv)

    h1 = _layer(h0, a0, a1, a2, inv8, wroot1, wrel1[0], wrel1[1], wrel1[2],
                b1.reshape(1, HID), da=48)

    h1v = h1.reshape(N_NODES * 8, 8)
    agg2 = _edge_aggregate(h1v, 8, src2d, slots2d, zeros)
    a0, a1, a2 = _agg_to_rel_major(agg2, 8)
    h2 = _layer(h1, a0, a1, a2, inv8, W_root2, W_rel2[0], W_rel2[1],
                W_rel2[2], b2.reshape(1, HID), da=64)

    return _pool_final(h2, batch, lin_W, lin_b2d=lin_b.reshape(1, N_CLS))


# speedup vs baseline: 6.1326x; 1.1006x over previous
import functools

import jax
import jax.numpy as jnp
from jax import lax
from jax.experimental import pallas as pl
from jax.experimental.pallas import tpu as pltpu
from jax.experimental.pallas import tpu_sc as plsc

N_NODES = 50000
N_EDGES = 800000
N_SHAPES = 64
N_COLORS = 32
EMB = 16
HID = 64
N_REL = 3
N_CLS = 4
N_GRAPHS = 512
IN_DIM = 2 * EMB + 1

BN = 5000
NB = N_NODES // BN
F32 = jnp.float32


def _embed_body(sid_ref, cid_ref, x_ref, semb_ref, cemb_ref, out_ref):
    sid = sid_ref[0]
    cid = cid_ref[0]
    iota_s = jax.lax.broadcasted_iota(jnp.int32, (N_SHAPES, BN), 0)
    iota_c = jax.lax.broadcasted_iota(jnp.int32, (N_COLORS, BN), 0)
    oh_s = (iota_s == sid).astype(F32)
    oh_c = (iota_c == cid).astype(F32)
    dn = (((0,), (0,)), ((), ()))
    s = jax.lax.dot_general(oh_s, semb_ref[...], dn, preferred_element_type=F32)
    c = jax.lax.dot_general(oh_c, cemb_ref[...], dn, preferred_element_type=F32)
    ones = jnp.ones((BN, 1), dtype=F32)
    pad = jnp.zeros((BN, HID - IN_DIM - 1), dtype=F32)
    out_ref[...] = jnp.concatenate([s, c, x_ref[...], ones, pad], axis=1)


def _layer_body(h_ref, a0_ref, a1_ref, a2_ref, inv_ref, wroot_ref,
                w0_ref, w1_ref, w2_ref, b_ref, out_ref):
    acc = jnp.dot(h_ref[...], wroot_ref[...], preferred_element_type=F32)
    inv = inv_ref[...]
    acc += jnp.dot(a0_ref[...] * inv[:, 0:1], w0_ref[...],
                   preferred_element_type=F32)
    acc += jnp.dot(a1_ref[...] * inv[:, 1:2], w1_ref[...],
                   preferred_element_type=F32)
    acc += jnp.dot(a2_ref[...] * inv[:, 2:3], w2_ref[...],
                   preferred_element_type=F32)
    out_ref[...] = jnp.maximum(acc + b_ref[...], 0.0)


def _layer(h, agg2d, inv, wroot, w0, w1, w2, b2d, da):
    nbr = N_NODES // BN
    spec_h = pl.BlockSpec((BN, HID), lambda i: (i, 0))
    spec_w = pl.BlockSpec((da, HID), lambda i: (0, 0))

    def aspec(r):
        return pl.BlockSpec((BN, da), lambda i, r=r: (r * nbr + i, 0))

    return pl.pallas_call(
        _layer_body,
        grid=(NB,),
        in_specs=[spec_h, aspec(0), aspec(1), aspec(2),
                  pl.BlockSpec((BN, 8), lambda i: (i, 0)),
                  pl.BlockSpec((HID, HID), lambda i: (0, 0)),
                  spec_w, spec_w, spec_w,
                  pl.BlockSpec((1, HID), lambda i: (0, 0))],
        out_specs=spec_h,
        out_shape=jax.ShapeDtypeStruct((N_NODES, HID), F32),
    )(h, agg2d, agg2d, agg2d, inv, wroot, w0, w1, w2, b2d)


def _pool_body(h_ref, batch_ref, linw_ref, linb_ref, out_ref, sums, cnts):
    i = pl.program_id(0)

    @pl.when(i == 0)
    def _():
        sums[...] = jnp.zeros_like(sums)
        cnts[...] = jnp.zeros_like(cnts)

    b = batch_ref[0]
    iota_g = jax.lax.broadcasted_iota(jnp.int32, (N_GRAPHS, BN), 0)
    oh = (iota_g == b).astype(F32)
    dn = (((1,), (0,)), ((), ()))
    sums[...] += jax.lax.dot_general(oh, h_ref[...], dn,
                                     preferred_element_type=F32)
    ones = jnp.ones((BN, 8), dtype=F32)
    cnts[...] += jax.lax.dot_general(oh, ones, dn, preferred_element_type=F32)

    @pl.when(i == NB - 1)
    def _():
        g = sums[...] / jnp.maximum(cnts[:, 0:1], 1.0)
        out_ref[...] = (jnp.dot(g, linw_ref[...], preferred_element_type=F32)
                        + linb_ref[...])


def _pool_final(h, batch, lin_W, lin_b2d):
    batch3 = batch.astype(jnp.int32).reshape(NB, 1, BN)
    return pl.pallas_call(
        _pool_body,
        grid=(NB,),
        in_specs=[
            pl.BlockSpec((BN, HID), lambda i: (i, 0)),
            pl.BlockSpec((1, 1, BN), lambda i: (i, 0, 0)),
            pl.BlockSpec((HID, N_CLS), lambda i: (0, 0)),
            pl.BlockSpec((1, N_CLS), lambda i: (0, 0)),
        ],
        out_specs=pl.BlockSpec((N_GRAPHS, N_CLS), lambda i: (0, 0)),
        out_shape=jax.ShapeDtypeStruct((N_GRAPHS, N_CLS), F32),
        scratch_shapes=[pltpu.VMEM((N_GRAPHS, HID), F32),
                        pltpu.VMEM((N_GRAPHS, 8), F32)],
    )(h, batch3, lin_W, lin_b2d)


E_PAD = 819200
CH_ROWS = E_PAD // 128
ROWS_PER_TILE = CH_ROWS // 16
SUPERS = ROWS_PER_TILE // 16
ACC_ROWS = N_REL * N_NODES + 16
STRIPE = ACC_ROWS // 16


@functools.lru_cache(maxsize=None)
def _make_agg(sf):
    s2 = sf // 2
    mesh = plsc.VectorSubcoreMesh(core_axis_name="c", subcore_axis_name="s")

    @functools.partial(
        pl.kernel,
        out_type=jax.ShapeDtypeStruct((ACC_ROWS, sf, 8), F32),
        mesh=mesh,
        scratch_types=[
            pltpu.VMEM_SHARED((ACC_ROWS, 8), F32),
            pltpu.VMEM((16, 128), jnp.int32),
            pltpu.VMEM((16, 128), jnp.int32),
            pltpu.VMEM((16, 128), jnp.int32),
            pltpu.VMEM((16, 128, 8), F32),
            pltpu.SemaphoreType.DMA,
            pltpu.SemaphoreType.DMA,
        ],
        compiler_params=pltpu.CompilerParams(use_tc_tiling_on_sc=False),
    )
    def agg_kernel(hview, src2d, slots2d, zeros, out,
                   acc, srcbuf, slotbuf, idxbuf, rows, gsem, ssem):
        c = lax.axis_index("c")
        t = lax.axis_index("s")
        r0 = t * ROWS_PER_TILE
        a0 = t * STRIPE
        for i in range(s2):
            s_id = c * s2 + i
            pltpu.sync_copy(zeros.at[pl.ds(a0, STRIPE)],
                            acc.at[pl.ds(a0, STRIPE)])
            plsc.subcore_barrier()

            def super_body(u, carry):
                row = r0 + u * 16
                pltpu.sync_copy(src2d.at[pl.ds(row, 16)], srcbuf)
                pltpu.sync_copy(slots2d.at[pl.ds(row, 16)], slotbuf)
                for g in range(16):
                    for q in range(8):
                        sl = pl.ds(q * 16, 16)
                        idxbuf[g, sl] = srcbuf[g, sl] * sf + s_id
                for g in range(16):
                    pltpu.async_copy(hview.at[idxbuf.at[g]], rows.at[g], gsem)
                for g in range(16):
                    pltpu.make_async_copy(hview.at[idxbuf.at[g]],
                                          rows.at[g], gsem).wait()
                for g in range(16):
                    pltpu.async_copy(rows.at[g], acc.at[slotbuf.at[g]],
                                     ssem, add=True)
                for g in range(16):
                    pltpu.make_async_copy(rows.at[g],
                                          acc.at[slotbuf.at[g]], ssem).wait()
                return carry

            lax.fori_loop(0, SUPERS, super_body, 0)
            plsc.subcore_barrier()
            pltpu.sync_copy(acc.at[pl.ds(a0, STRIPE)],
                            out.at[pl.ds(a0, STRIPE), s_id])
            plsc.subcore_barrier()

    return agg_kernel


def _edge_aggregate(hview, sf, src2d, slots2d, zeros):
    return _make_agg(sf)(hview, src2d, slots2d, zeros)


def kernel(x, shape_id, color_id, edge_index, edge_type, batch, shape_emb,
           color_emb, W_rel1, W_root1, b1, W_rel2, W_root2, b2, lin_W, lin_b):
    src = edge_index[0].astype(jnp.int32)
    dst = edge_index[1].astype(jnp.int32)
    etype = edge_type.astype(jnp.int32)

    n_pad = E_PAD - N_EDGES
    pad_iota = jnp.arange(n_pad, dtype=jnp.int32)
    src_p = jnp.concatenate([src, pad_iota % N_NODES])
    slots_p = jnp.concatenate([etype * N_NODES + dst,
                               N_REL * N_NODES + (pad_iota % 16)])
    src2d = src_p.reshape(CH_ROWS, 128)
    slots2d = slots_p.reshape(CH_ROWS, 128)
    zeros = jnp.zeros((ACC_ROWS, 8), F32)

    wroot1 = jnp.zeros((HID, HID), F32).at[:IN_DIM].set(W_root1)
    wrel1 = jnp.zeros((N_REL, 48, HID), F32).at[:, :IN_DIM].set(W_rel1)

    sid3 = shape_id.astype(jnp.int32).reshape(NB, 1, BN)
    cid3 = color_id.astype(jnp.int32).reshape(NB, 1, BN)
    h0 = pl.pallas_call(
        _embed_body,
        grid=(NB,),
        in_specs=[
            pl.BlockSpec((1, 1, BN), lambda i: (i, 0, 0)),
            pl.BlockSpec((1, 1, BN), lambda i: (i, 0, 0)),
            pl.BlockSpec((BN, 1), lambda i: (i, 0)),
            pl.BlockSpec((N_SHAPES, EMB), lambda i: (0, 0)),
            pl.BlockSpec((N_COLORS, EMB), lambda i: (0, 0)),
        ],
        out_specs=pl.BlockSpec((BN, HID), lambda i: (i, 0)),
        out_shape=jax.ShapeDtypeStruct((N_NODES, HID), F32),
    )(sid3, cid3, x, shape_emb, color_emb)

    h0v = h0[:, :48].reshape(N_NODES * 6, 8)
    agg1 = _edge_aggregate(h0v, 6, src2d, slots2d, zeros)
    agg1_2d = agg1.reshape(ACC_ROWS, 48)
    cnt = agg1_2d[:N_REL * N_NODES, IN_DIM].reshape(N_REL, N_NODES)
    inv = (1.0 / jnp.maximum(cnt, 1.0)).T
    inv8 = jnp.zeros((N_NODES, 8), F32).at[:, :N_REL].set(inv)

    h1 = _layer(h0, agg1_2d, inv8, wroot1, wrel1[0], wrel1[1], wrel1[2],
                b1.reshape(1, HID), da=48)

    h1v = h1.reshape(N_NODES * 8, 8)
    agg2 = _edge_aggregate(h1v, 8, src2d, slots2d, zeros)
    h2 = _layer(h1, agg2.reshape(ACC_ROWS, 64), inv8, W_root2, W_rel2[0],
                W_rel2[1], W_rel2[2], b2.reshape(1, HID), da=64)

    return _pool_final(h2, batch, lin_W, lin_b2d=lin_b.reshape(1, N_CLS))


# speedup vs baseline: 6.9255x; 1.1293x over previous
import functools

import jax
import jax.numpy as jnp
from jax import lax
from jax.experimental import pallas as pl
from jax.experimental.pallas import tpu as pltpu
from jax.experimental.pallas import tpu_sc as plsc

N_NODES = 50000
N_EDGES = 800000
N_SHAPES = 64
N_COLORS = 32
EMB = 16
HID = 64
N_REL = 3
N_CLS = 4
N_GRAPHS = 512
IN_DIM = 2 * EMB + 1

BN = 5000
NB = N_NODES // BN
F32 = jnp.float32


def _embed_body(sid_ref, cid_ref, x_ref, semb_ref, cemb_ref, out_ref):
    sid = sid_ref[0]
    cid = cid_ref[0]
    iota_s = jax.lax.broadcasted_iota(jnp.int32, (N_SHAPES, BN), 0)
    iota_c = jax.lax.broadcasted_iota(jnp.int32, (N_COLORS, BN), 0)
    oh_s = (iota_s == sid).astype(F32)
    oh_c = (iota_c == cid).astype(F32)
    dn = (((0,), (0,)), ((), ()))
    s = jax.lax.dot_general(oh_s, semb_ref[...], dn, preferred_element_type=F32)
    c = jax.lax.dot_general(oh_c, cemb_ref[...], dn, preferred_element_type=F32)
    ones = jnp.ones((BN, 1), dtype=F32)
    pad = jnp.zeros((BN, 48 - IN_DIM - 1), dtype=F32)
    out_ref[...] = jnp.concatenate([s, c, x_ref[...], ones, pad], axis=1)


def _layer_body(h_ref, a0_ref, a1_ref, a2_ref, inv_ref, wroot_ref,
                w0_ref, w1_ref, w2_ref, b_ref, out_ref):
    acc = jnp.dot(h_ref[...], wroot_ref[...], preferred_element_type=F32)
    inv = inv_ref[...]
    acc += jnp.dot(a0_ref[...] * inv[:, 0:1], w0_ref[...],
                   preferred_element_type=F32)
    acc += jnp.dot(a1_ref[...] * inv[:, 1:2], w1_ref[...],
                   preferred_element_type=F32)
    acc += jnp.dot(a2_ref[...] * inv[:, 2:3], w2_ref[...],
                   preferred_element_type=F32)
    out_ref[...] = jnp.maximum(acc + b_ref[...], 0.0)


def _layer(h, agg2d, inv, wroot, w0, w1, w2, b2d, da, dh):
    nbr = N_NODES // BN
    spec_h = pl.BlockSpec((BN, dh), lambda i: (i, 0))
    spec_w = pl.BlockSpec((da, HID), lambda i: (0, 0))

    def aspec(r):
        return pl.BlockSpec((BN, da), lambda i, r=r: (r * nbr + i, 0))

    return pl.pallas_call(
        _layer_body,
        grid=(NB,),
        in_specs=[spec_h, aspec(0), aspec(1), aspec(2),
                  pl.BlockSpec((BN, 8), lambda i: (i, 0)),
                  pl.BlockSpec((dh, HID), lambda i: (0, 0)),
                  spec_w, spec_w, spec_w,
                  pl.BlockSpec((1, HID), lambda i: (0, 0))],
        out_specs=pl.BlockSpec((BN, HID), lambda i: (i, 0)),
        out_shape=jax.ShapeDtypeStruct((N_NODES, HID), F32),
    )(h, agg2d, agg2d, agg2d, inv, wroot, w0, w1, w2, b2d)


def _pool_body(h_ref, batch_ref, linw_ref, linb_ref, out_ref, sums, cnts):
    i = pl.program_id(0)

    @pl.when(i == 0)
    def _():
        sums[...] = jnp.zeros_like(sums)
        cnts[...] = jnp.zeros_like(cnts)

    b = batch_ref[0]
    iota_g = jax.lax.broadcasted_iota(jnp.int32, (N_GRAPHS, BN), 0)
    oh = (iota_g == b).astype(F32)
    dn = (((1,), (0,)), ((), ()))
    sums[...] += jax.lax.dot_general(oh, h_ref[...], dn,
                                     preferred_element_type=F32)
    ones = jnp.ones((BN, 8), dtype=F32)
    cnts[...] += jax.lax.dot_general(oh, ones, dn, preferred_element_type=F32)

    @pl.when(i == NB - 1)
    def _():
        g = sums[...] / jnp.maximum(cnts[:, 0:1], 1.0)
        out_ref[...] = (jnp.dot(g, linw_ref[...], preferred_element_type=F32)
                        + linb_ref[...])


def _pool_final(h, batch, lin_W, lin_b2d):
    batch3 = batch.astype(jnp.int32).reshape(NB, 1, BN)
    return pl.pallas_call(
        _pool_body,
        grid=(NB,),
        in_specs=[
            pl.BlockSpec((BN, HID), lambda i: (i, 0)),
            pl.BlockSpec((1, 1, BN), lambda i: (i, 0, 0)),
            pl.BlockSpec((HID, N_CLS), lambda i: (0, 0)),
            pl.BlockSpec((1, N_CLS), lambda i: (0, 0)),
        ],
        out_specs=pl.BlockSpec((N_GRAPHS, N_CLS), lambda i: (0, 0)),
        out_shape=jax.ShapeDtypeStruct((N_GRAPHS, N_CLS), F32),
        scratch_shapes=[pltpu.VMEM((N_GRAPHS, HID), F32),
                        pltpu.VMEM((N_GRAPHS, 8), F32)],
    )(h, batch3, lin_W, lin_b2d)


E_PAD = 851968
CH_ROWS = E_PAD // 128
ROWS_PER_TILE = CH_ROWS // 16
SUPERS = ROWS_PER_TILE // 16
ACC_ROWS = N_REL * N_NODES + 16
STRIPE = ACC_ROWS // 16


@functools.lru_cache(maxsize=None)
def _make_agg(sf):
    s2 = sf // 2
    mesh = plsc.VectorSubcoreMesh(core_axis_name="c", subcore_axis_name="s")

    @functools.partial(
        pl.kernel,
        out_type=jax.ShapeDtypeStruct((ACC_ROWS, sf, 8), F32),
        mesh=mesh,
        scratch_types=[
            pltpu.VMEM_SHARED((ACC_ROWS, 8), F32),
            pltpu.VMEM((2, 16, 128), jnp.int32),
            pltpu.VMEM((2, 16, 128), jnp.int32),
            pltpu.VMEM((2, 16, 128), jnp.int32),
            pltpu.VMEM((2, 16, 128, 8), F32),
            pltpu.SemaphoreType.DMA,
            pltpu.SemaphoreType.DMA,
        ],
        compiler_params=pltpu.CompilerParams(use_tc_tiling_on_sc=False),
    )
    def agg_kernel(hview, src2d, slots2d, zeros, out,
                   acc, srcbuf, slotbuf, idxbuf, rows, gsem, ssem):
        c = lax.axis_index("c")
        t = lax.axis_index("s")
        r0 = t * ROWS_PER_TILE
        a0 = t * STRIPE
        pairs = SUPERS // 2

        for i in range(s2):
            s_id = c * s2 + i

            def stage(p, u):
                row = r0 + u * 16
                pltpu.sync_copy(src2d.at[pl.ds(row, 16)], srcbuf.at[p])
                pltpu.sync_copy(slots2d.at[pl.ds(row, 16)], slotbuf.at[p])
                for g in range(16):
                    for q in range(8):
                        sl = pl.ds(q * 16, 16)
                        idxbuf[p, g, sl] = srcbuf[p, g, sl] * sf + s_id

            def fire_gathers(p):
                for g in range(16):
                    pltpu.async_copy(hview.at[idxbuf.at[p].at[g]],
                                     rows.at[p].at[g], gsem)

            def drain_gathers(p):
                for g in range(16):
                    pltpu.make_async_copy(hview.at[idxbuf.at[p].at[g]],
                                          rows.at[p].at[g], gsem).wait()

            def fire_scatters(p):
                for g in range(16):
                    pltpu.async_copy(rows.at[p].at[g],
                                     acc.at[slotbuf.at[p].at[g]],
                                     ssem, add=True)

            def drain_scatters(p):
                for g in range(16):
                    pltpu.make_async_copy(rows.at[p].at[g],
                                          acc.at[slotbuf.at[p].at[g]],
                                          ssem).wait()

            pltpu.sync_copy(zeros.at[pl.ds(a0, STRIPE)],
                            acc.at[pl.ds(a0, STRIPE)])
            stage(0, 0)
            fire_gathers(0)
            plsc.subcore_barrier()

            def pair_body(v, carry):
                stage(1, 2 * v + 1)
                drain_gathers(0)
                fire_scatters(0)
                fire_gathers(1)
                drain_scatters(0)

                @pl.when(v < pairs - 1)
                def _():
                    stage(0, 2 * v + 2)

                drain_gathers(1)
                fire_scatters(1)

                @pl.when(v < pairs - 1)
                def _():
                    fire_gathers(0)

                drain_scatters(1)
                return carry

            lax.fori_loop(0, pairs, pair_body, 0)
            plsc.subcore_barrier()
            pltpu.sync_copy(acc.at[pl.ds(a0, STRIPE)],
                            out.at[pl.ds(a0, STRIPE), s_id])
            plsc.subcore_barrier()

    return agg_kernel


def _edge_aggregate(hview, sf, src2d, slots2d, zeros):
    return _make_agg(sf)(hview, src2d, slots2d, zeros)


def kernel(x, shape_id, color_id, edge_index, edge_type, batch, shape_emb,
           color_emb, W_rel1, W_root1, b1, W_rel2, W_root2, b2, lin_W, lin_b):
    src = edge_index[0].astype(jnp.int32)
    dst = edge_index[1].astype(jnp.int32)
    etype = edge_type.astype(jnp.int32)

    n_pad = E_PAD - N_EDGES
    pad_iota = jnp.arange(n_pad, dtype=jnp.int32)
    src_p = jnp.concatenate([src, pad_iota % N_NODES])
    slots_p = jnp.concatenate([etype * N_NODES + dst,
                               N_REL * N_NODES + (pad_iota % 16)])
    src2d = src_p.reshape(CH_ROWS, 128)
    slots2d = slots_p.reshape(CH_ROWS, 128)
    zeros = jnp.zeros((ACC_ROWS, 8), F32)

    wroot1 = jnp.zeros((48, HID), F32).at[:IN_DIM].set(W_root1)
    wrel1 = jnp.zeros((N_REL, 48, HID), F32).at[:, :IN_DIM].set(W_rel1)

    sid3 = shape_id.astype(jnp.int32).reshape(NB, 1, BN)
    cid3 = color_id.astype(jnp.int32).reshape(NB, 1, BN)
    h0 = pl.pallas_call(
        _embed_body,
        grid=(NB,),
        in_specs=[
            pl.BlockSpec((1, 1, BN), lambda i: (i, 0, 0)),
            pl.BlockSpec((1, 1, BN), lambda i: (i, 0, 0)),
            pl.BlockSpec((BN, 1), lambda i: (i, 0)),
            pl.BlockSpec((N_SHAPES, EMB), lambda i: (0, 0)),
            pl.BlockSpec((N_COLORS, EMB), lambda i: (0, 0)),
        ],
        out_specs=pl.BlockSpec((BN, 48), lambda i: (i, 0)),
        out_shape=jax.ShapeDtypeStruct((N_NODES, 48), F32),
    )(sid3, cid3, x, shape_emb, color_emb)

    h0v = h0.reshape(N_NODES * 6, 8)
    agg1 = _edge_aggregate(h0v, 6, src2d, slots2d, zeros)
    agg1_2d = agg1.reshape(ACC_ROWS, 48)
    cnt = agg1_2d[:N_REL * N_NODES, IN_DIM].reshape(N_REL, N_NODES)
    inv = (1.0 / jnp.maximum(cnt, 1.0)).T
    inv8 = jnp.zeros((N_NODES, 8), F32).at[:, :N_REL].set(inv)

    h1 = _layer(h0, agg1_2d, inv8, wroot1, wrel1[0], wrel1[1], wrel1[2],
                b1.reshape(1, HID), da=48, dh=48)

    h1v = h1.reshape(N_NODES * 8, 8)
    agg2 = _edge_aggregate(h1v, 8, src2d, slots2d, zeros)
    h2 = _layer(h1, agg2.reshape(ACC_ROWS, 64), inv8, W_root2, W_rel2[0],
                W_rel2[1], W_rel2[2], b2.reshape(1, HID), da=64, dh=64)

    return _pool_final(h2, batch, lin_W, lin_b2d=lin_b.reshape(1, N_CLS))
